# Initial kernel scaffold; baseline (speedup 1.0000x reference)
import jax
import jax.numpy as jnp
from jax.experimental import pallas as pl


def kernel(x, weight):
    raise NotImplementedError("write your pallas kernel here")


# speedup vs baseline: 3.7343x; 3.7343x over previous
import functools

import jax
import jax.numpy as jnp
from jax.experimental import pallas as pl

_T = 32768
_DIM = 2048
_N_EXPERTS = 8
_N_GROUPS = 2
_GROUP_SIZE = _N_EXPERTS // _N_GROUPS
_ROUTE_SCALE = 2.5
_BLOCK = 1024


def _gate_block(x_ref, w_ref, wout_ref, iout_ref):
    x = x_ref[...]
    w = w_ref[...]
    s = jax.lax.dot_general(
        x, w, (((1,), (1,)), ((), ())), preferred_element_type=jnp.float32
    )
    s = jax.nn.sigmoid(s)

    col = jax.lax.broadcasted_iota(jnp.int32, s.shape, 1)
    in_g0 = col < _GROUP_SIZE
    neg = jnp.float32(-1.0)

    g0 = jnp.max(jnp.where(in_g0, s, neg), axis=1, keepdims=True)
    g1 = jnp.max(jnp.where(in_g0, neg, s), axis=1, keepdims=True)
    chosen0 = g0 >= g1

    keep = in_g0 == chosen0
    m = jnp.where(keep, s, neg)

    v1 = jnp.max(m, axis=1, keepdims=True)
    i1 = jnp.min(jnp.where(m == v1, col, _N_EXPERTS), axis=1, keepdims=True)
    m2 = jnp.where(col == i1, neg, m)
    v2 = jnp.max(m2, axis=1, keepdims=True)
    i2 = jnp.min(jnp.where(m2 == v2, col, _N_EXPERTS), axis=1, keepdims=True)

    scale = _ROUTE_SCALE / (v1 + v2)
    wout_ref[:, 0:1] = v1 * scale
    wout_ref[:, 1:2] = v2 * scale
    iout_ref[:, 0:1] = i1
    iout_ref[:, 1:2] = i2


@jax.jit
def kernel(x, weight):
    n_blocks = _T // _BLOCK
    weights, indices = pl.pallas_call(
        _gate_block,
        grid=(n_blocks,),
        in_specs=[
            pl.BlockSpec((_BLOCK, _DIM), lambda i: (i, 0)),
            pl.BlockSpec((_N_EXPERTS, _DIM), lambda i: (0, 0)),
        ],
        out_specs=[
            pl.BlockSpec((_BLOCK, 2), lambda i: (i, 0)),
            pl.BlockSpec((_BLOCK, 2), lambda i: (i, 0)),
        ],
        out_shape=[
            jax.ShapeDtypeStruct((_T, 2), jnp.float32),
            jax.ShapeDtypeStruct((_T, 2), jnp.int32),
        ],
    )(x, weight)
    return weights, indices


# speedup vs baseline: 3.9728x; 1.0639x over previous
import functools

import jax
import jax.numpy as jnp
from jax.experimental import pallas as pl

_T = 32768
_DIM = 2048
_N_EXPERTS = 8
_N_GROUPS = 2
_GROUP_SIZE = _N_EXPERTS // _N_GROUPS
_ROUTE_SCALE = 2.5
_BLOCK = 2048


def _gate_block(x_ref, w_ref, wout_ref, iout_ref):
    x = x_ref[...]
    w = w_ref[...]
    s = jax.lax.dot_general(
        x, w, (((1,), (1,)), ((), ())), preferred_element_type=jnp.float32
    )
    s = jax.nn.sigmoid(s)

    col = jax.lax.broadcasted_iota(jnp.int32, s.shape, 1)
    in_g0 = col < _GROUP_SIZE
    neg = jnp.float32(-1.0)

    g0 = jnp.max(jnp.where(in_g0, s, neg), axis=1, keepdims=True)
    g1 = jnp.max(jnp.where(in_g0, neg, s), axis=1, keepdims=True)
    chosen0 = g0 >= g1

    keep = in_g0 == chosen0
    m = jnp.where(keep, s, neg)

    v1 = jnp.max(m, axis=1, keepdims=True)
    i1 = jnp.min(jnp.where(m == v1, col, _N_EXPERTS), axis=1, keepdims=True)
    m2 = jnp.where(col == i1, neg, m)
    v2 = jnp.max(m2, axis=1, keepdims=True)
    i2 = jnp.min(jnp.where(m2 == v2, col, _N_EXPERTS), axis=1, keepdims=True)

    scale = _ROUTE_SCALE / (v1 + v2)
    wout_ref[:, 0:1] = v1 * scale
    wout_ref[:, 1:2] = v2 * scale
    iout_ref[:, 0:1] = i1
    iout_ref[:, 1:2] = i2


@jax.jit
def kernel(x, weight):
    n_blocks = _T // _BLOCK
    weights, indices = pl.pallas_call(
        _gate_block,
        grid=(n_blocks,),
        in_specs=[
            pl.BlockSpec((_BLOCK, _DIM), lambda i: (i, 0)),
            pl.BlockSpec((_N_EXPERTS, _DIM), lambda i: (0, 0)),
        ],
        out_specs=[
            pl.BlockSpec((_BLOCK, 2), lambda i: (i, 0)),
            pl.BlockSpec((_BLOCK, 2), lambda i: (i, 0)),
        ],
        out_shape=[
            jax.ShapeDtypeStruct((_T, 2), jnp.float32),
            jax.ShapeDtypeStruct((_T, 2), jnp.int32),
        ],
    )(x, weight)
    return weights, indices
